# baseline (speedup 1.0000x reference)
import functools

import numpy as np
import jax
import jax.numpy as jnp
from jax import lax
from jax.experimental import pallas as pl
from jax.experimental.pallas import tpu as pltpu
from jax.experimental.pallas import tpu_sc as plsc

_NC = 2
_NS = 16
_NW = _NC * _NS
_L = 16
_BL = 128


def _positional_encoding(length: int, d_model: int) -> np.ndarray:
    positions = np.arange(length)[:, None]
    dims = np.arange(d_model)[None, :]
    angle_rates = 1.0 / np.power(10000.0, 2 * (dims // 2) / np.float32(d_model))
    angle_rads = positions * angle_rates
    pos = np.zeros((length, d_model), dtype=np.float32)
    pos[:, 0::2] = np.sin(angle_rads[:, 0::2])
    pos[:, 1::2] = np.cos(angle_rads[:, 1::2])
    return pos


def _make_sc_kernel(B: int, S: int, D: int, DP: int, OP: int):
    scale = float(np.sqrt(np.float32(D)))
    groups = D // _L

    mesh = plsc.VectorSubcoreMesh(core_axis_name="c", subcore_axis_name="s")

    @functools.partial(
        pl.kernel,
        mesh=mesh,
        out_type=jax.ShapeDtypeStruct((B, S, OP), jnp.float32),
        compiler_params=pltpu.CompilerParams(use_tc_tiling_on_sc=False),
        scratch_types=[
            pltpu.VMEM((S, _BL), jnp.int32),
            pltpu.VMEM((_BL, DP), jnp.float32),
            pltpu.VMEM((_BL, DP), jnp.float32),
            pltpu.VMEM((_BL, DP), jnp.float32),
            pltpu.VMEM((_BL, D), jnp.float32),
            pltpu.VMEM((_BL, D), jnp.float32),
            pltpu.VMEM((S, D), jnp.float32),
            pltpu.SemaphoreType.DMA,
            pltpu.SemaphoreType.DMA,
            pltpu.SemaphoreType.DMA,
            pltpu.SemaphoreType.DMA,
            pltpu.SemaphoreType.DMA,
        ],
    )
    def k(xt_hbm, table_hbm, pos_hbm, out_hbm,
          idx_v, b0, b1, b2, ob0, ob1, pos_v,
          g0, g1, g2, o0, o1):
        wid = lax.axis_index("s") * _NC + lax.axis_index("c")
        bufs = (b0, b1, b2)
        obufs = (ob0, ob1)
        gsems = (g0, g1, g2)
        osems = (o0, o1)

        pltpu.sync_copy(pos_hbm, pos_v)
        pltpu.sync_copy(xt_hbm.at[:, pl.ds(wid * _BL, _BL)], idx_v)
        base_b = wid * _BL

        def gather(s, slot):
            pltpu.async_copy(
                table_hbm.at[idx_v.at[s]], bufs[slot], gsems[slot])

        def wait_gather(s, slot):
            pltpu.make_async_copy(
                table_hbm.at[idx_v.at[s]], bufs[slot], gsems[slot]).wait()

        def out_copy(s, oslot):
            return pltpu.make_async_copy(
                obufs[oslot],
                out_hbm.at[pl.ds(base_b, _BL), s, pl.ds(0, D)],
                osems[oslot])

        gather(0, 0)
        gather(1, 1)
        gather(2, 2)

        def step(s, gslot, oslot):
            buf = bufs[gslot]
            obuf = obufs[oslot]
            wait_gather(s, gslot)

            @pl.when(s >= 2)
            def _():
                out_copy(s - 2, oslot).wait()

            ps = [pos_v[s, pl.ds(g * _L, _L)] for g in range(groups)]

            def row_body(r, carry):
                for g in range(groups):
                    sl = pl.ds(g * _L, _L)
                    obuf[r, sl] = buf[r, sl] * scale + ps[g]
                return carry

            lax.fori_loop(0, _BL, row_body, 0, unroll=8)
            out_copy(s, oslot).start()

            @pl.when(s + 3 < S)
            def _():
                gather(s + 3, gslot)

        def hex_body(j, carry):
            for kk in range(6):
                s = 6 * j + kk

                @pl.when(s < S)
                def _():
                    step(s, kk % 3, kk % 2)

            return carry

        lax.fori_loop(0, (S + 5) // 6, hex_body, 0)
        out_copy(S - 2, (S - 2) % 2).wait()
        out_copy(S - 1, (S - 1) % 2).wait()

    return k


def kernel(x, table):
    B, S = x.shape
    V, D = table.shape
    DP = 2 * D
    OP = 2 * D
    pos = jnp.asarray(_positional_encoding(S, D))
    table_p = jnp.pad(table, ((0, 0), (0, DP - D)))
    xt = x.T.astype(jnp.int32)
    k = _make_sc_kernel(B, S, D, DP, OP)
    out = k(xt, table_p, pos)
    return out[:, :, :D]
